# baseline (speedup 1.0000x reference)
import functools

import jax
import jax.numpy as jnp
from jax import lax
from jax.experimental import pallas as pl
from jax.experimental.pallas import tpu as pltpu
from jax.experimental.pallas import tpu_sc as plsc

B = 16
C = 3
H = 512
W = 512
HW = H * W
N = 16384
L = 16

NF = N // 2
FPW = NF // 2
KF = 1024
PW = 88
PPLANE = PW * PW
PMARG = 40

K = 1024
SUB = 256
NPCH = (N // 2) // K // 2

_mesh = plsc.VectorSubcoreMesh(core_axis_name="c", subcore_axis_name="s")


@functools.partial(
    pl.kernel,
    out_type=jax.ShapeDtypeStruct((B * C * N,), jnp.float32),
    mesh=_mesh,
    compiler_params=pltpu.CompilerParams(needs_layout_passes=False),
    scratch_types=[
        pltpu.VMEM((C * PPLANE,), jnp.float32),
        pltpu.VMEM((FPW,), jnp.float32),
        pltpu.VMEM((FPW,), jnp.float32),
        pltpu.VMEM((2 * C * KF,), jnp.float32),
        pltpu.VMEM((NPCH * K,), jnp.float32),
        pltpu.VMEM((NPCH * K,), jnp.float32),
        pltpu.VMEM((3 * 4 * K,), jnp.float32),
        pltpu.VMEM((3 * 4 * K,), jnp.int32),
        pltpu.VMEM((3 * 12 * K,), jnp.float32),
        pltpu.VMEM((2 * 3 * K,), jnp.float32),
        pltpu.VMEM((5 * L,), jnp.float32),
        pltpu.SemaphoreType.DMA,
        pltpu.SemaphoreType.DMA,
        pltpu.SemaphoreType.DMA,
        pltpu.SemaphoreType.DMA,
        pltpu.SemaphoreType.DMA,
        pltpu.SemaphoreType.DMA,
        pltpu.SemaphoreType.DMA,
    ],
)
def _sampler(img_hbm, gx_hbm, gy_hbm, fs_hbm, flx_hbm, fly_hbm, pxy_hbm,
             out_hbm,
             patch, gxf, gyf, outf, gx_v, gy_v, wbuf, idxbuf, valbuf, outbuf,
             parbuf, semp, semgr, semg0, semg1, semg2, semo0, semo1):
    core = lax.axis_index("c")
    b = lax.axis_index("s")
    semg = (semg0, semg1, semg2)
    semo = (semo0, semo1)

    pltpu.sync_copy(fs_hbm.at[pl.ds(b * L, L)], parbuf.at[pl.ds(0, L)])
    pltpu.sync_copy(flx_hbm.at[pl.ds(b * L, L)], parbuf.at[pl.ds(L, L)])
    pltpu.sync_copy(fly_hbm.at[pl.ds(b * L, L)], parbuf.at[pl.ds(2 * L, L)])
    pltpu.sync_copy(pxy_hbm.at[pl.ds(b * 2 * L, 2 * L)],
                    parbuf.at[pl.ds(3 * L, 2 * L)])
    scale = parbuf[pl.ds(0, L)] * (W / 2.0)
    tx = parbuf[pl.ds(L, L)] * (W / 2.0) + (W - 1) / 2.0
    ty = parbuf[pl.ds(2 * L, L)] * (H / 2.0) + (H - 1) / 2.0
    px0v = parbuf[pl.ds(3 * L, L)].astype(jnp.int32)
    py0v = parbuf[pl.ds(4 * L, L)].astype(jnp.int32)
    px0 = pl.multiple_of(px0v[0], 8)
    py0 = pl.multiple_of(py0v[0], 8)
    pbase = py0v * PW + px0v
    plane0 = b * (C * HW)
    out0 = b * (C * N)

    def floor16(x):
        return (x + 128.0).astype(jnp.int32) - 128

    def fire_patch():
        cps = []
        for c in range(C):
            base = plane0 + c * HW + py0 * W + px0
            for r in range(PW):
                cps.append(pltpu.async_copy(
                    img_hbm.at[pl.ds(base + r * W, PW)],
                    patch.at[pl.ds(c * PPLANE + r * PW, PW)],
                    semp,
                ))
        return cps

    def sub_base(t, i):
        return NF + (2 * ((t * K + i * SUB) // SUB) + core) * SUB

    def fire_grid():
        cps = []
        for t in range(NPCH):
            for i in range(K // SUB):
                srcs = pl.ds(sub_base(t, i), SUB)
                dst = pl.ds(t * K + i * SUB, SUB)
                cps.append(pltpu.async_copy(gx_hbm.at[srcs], gx_v.at[dst], semgr))
                cps.append(pltpu.async_copy(gy_hbm.at[srcs], gy_v.at[dst], semgr))
        s0 = core * FPW
        cps.append(pltpu.async_copy(gx_hbm.at[pl.ds(s0, FPW)], gxf, semgr))
        cps.append(pltpu.async_copy(gy_hbm.at[pl.ds(s0, FPW)], gyf, semgr))
        return cps

    def compute_idx(rg, t):
        g0 = t * K
        w0 = rg * 4 * K

        @plsc.parallel_loop(0, K, step=L, unroll=4)
        def body(off):
            gxv = gx_v[pl.ds(g0 + off, L)]
            gyv = gy_v[pl.ds(g0 + off, L)]
            ix = gxv * scale + tx
            iy = gyv * scale + ty
            ix0 = floor16(ix)
            iy0 = floor16(iy)
            wx1 = ix - ix0.astype(jnp.float32)
            wx0 = 1.0 - wx1
            wy1 = iy - iy0.astype(jnp.float32)
            wy0 = 1.0 - wy1
            vx0 = (ix0 >= 0) & (ix0 <= W - 1)
            vx1 = (ix0 >= -1) & (ix0 <= W - 2)
            vy0 = (iy0 >= 0) & (iy0 <= H - 1)
            vy1 = (iy0 >= -1) & (iy0 <= H - 2)
            fzero = jnp.zeros((L,), jnp.float32)
            wbuf[pl.ds(w0 + 0 * K + off, L)] = jnp.where(vy0 & vx0, wy0 * wx0, fzero)
            wbuf[pl.ds(w0 + 1 * K + off, L)] = jnp.where(vy0 & vx1, wy0 * wx1, fzero)
            wbuf[pl.ds(w0 + 2 * K + off, L)] = jnp.where(vy1 & vx0, wy1 * wx0, fzero)
            wbuf[pl.ds(w0 + 3 * K + off, L)] = jnp.where(vy1 & vx1, wy1 * wx1, fzero)
            ixc0 = jnp.clip(ix0, 0, W - 1)
            ixc1 = jnp.clip(ix0 + 1, 0, W - 1)
            iyc0 = jnp.clip(iy0, 0, H - 1)
            iyc1 = jnp.clip(iy0 + 1, 0, H - 1)
            dx = ixc1 - ixc0
            i00 = iyc0 * W + ixc0
            i10 = iyc1 * W + ixc0
            idxbuf[pl.ds(w0 + 0 * K + off, L)] = i00
            idxbuf[pl.ds(w0 + 1 * K + off, L)] = i00 + dx
            idxbuf[pl.ds(w0 + 2 * K + off, L)] = i10
            idxbuf[pl.ds(w0 + 3 * K + off, L)] = i10 + dx

    def fire_gathers(ph):
        cps = []
        for q in range(4):
            idx_ref = idxbuf.at[pl.ds((ph * 4 + q) * K, K)]
            for c in range(C):
                plane = img_hbm.at[pl.ds(plane0 + c * HW, HW)]
                dst = valbuf.at[pl.ds((ph * 12 + q * C + c) * K, K)]
                cps.append(pltpu.async_copy(plane.at[idx_ref], dst, semg[ph]))
        return cps

    def combine(rg, ph):
        w0 = rg * 4 * K
        v0 = rg * 12 * K
        o0 = ph * 3 * K

        @plsc.parallel_loop(0, K, step=L, unroll=4)
        def body(off):
            ws = [wbuf[pl.ds(w0 + q * K + off, L)] for q in range(4)]
            for c in range(C):
                acc = ws[0] * valbuf[pl.ds(v0 + (0 * C + c) * K + off, L)]
                acc = acc + ws[1] * valbuf[pl.ds(v0 + (1 * C + c) * K + off, L)]
                acc = acc + ws[2] * valbuf[pl.ds(v0 + (2 * C + c) * K + off, L)]
                acc = acc + ws[3] * valbuf[pl.ds(v0 + (3 * C + c) * K + off, L)]
                outbuf[pl.ds(o0 + c * K + off, L)] = acc

    def write_out(ph, t):
        cps = []
        for c in range(C):
            for i in range(K // SUB):
                cps.append(pltpu.async_copy(
                    outbuf.at[pl.ds((ph * 3 + c) * K + i * SUB, SUB)],
                    out_hbm.at[pl.ds(out0 + c * N + sub_base(t, i), SUB)],
                    semo[ph],
                ))
        return cps

    def fovea_chunk(ph, u):
        s0 = core * FPW + u * KF
        g0 = u * KF
        o0 = ph * C * KF

        @plsc.parallel_loop(0, KF, step=L, unroll=8)
        def body(off):
            gxv = gxf[pl.ds(g0 + off, L)]
            gyv = gyf[pl.ds(g0 + off, L)]
            ix = gxv * scale + tx
            iy = gyv * scale + ty
            ix0 = floor16(ix)
            iy0 = floor16(iy)
            wx1 = ix - ix0.astype(jnp.float32)
            wx0 = 1.0 - wx1
            wy1 = iy - iy0.astype(jnp.float32)
            wy0 = 1.0 - wy1
            w00 = wy0 * wx0
            w01 = wy0 * wx1
            w10 = wy1 * wx0
            w11 = wy1 * wx1
            ip = iy0 * PW + ix0 - pbase
            vs = []
            for c in range(C):
                p00 = ip + c * PPLANE
                vs.append((plsc.load_gather(patch, [p00]),
                           plsc.load_gather(patch, [p00 + 1]),
                           plsc.load_gather(patch, [p00 + PW]),
                           plsc.load_gather(patch, [p00 + PW + 1])))
            for c in range(C):
                v00, v01, v10, v11 = vs[c]
                acc = (w00 * v00 + w01 * v01) + (w10 * v10 + w11 * v11)
                outf[pl.ds(o0 + c * KF + off, L)] = acc

        return [
            pltpu.async_copy(
                outf.at[pl.ds((ph * C + c) * KF, KF)],
                out_hbm.at[pl.ds(out0 + c * N + s0, KF)],
                semo[ph],
            )
            for c in range(C)
        ]

    with jax.named_scope("fire_patch"):
        patch_cps = fire_patch()
        grid_cps = fire_grid()
        for cp in grid_cps:
            cp.wait()
    with jax.named_scope("idx012"):
        gq = []
        for r in range(3):
            compute_idx(r, r)
            gq.append(fire_gathers(r))

    with jax.named_scope("patch_wait"):
        for cp in patch_cps:
            cp.wait()
    pending = [None, None]
    with jax.named_scope("fovea"):
        for u in range(FPW // KF):
            ph = u % 2
            if pending[ph] is not None:
                for cp in pending[ph]:
                    cp.wait()
            pending[ph] = fovea_chunk(ph, u)

    for t in range(NPCH):
        rg = t % 3
        ph = t % 2
        with jax.named_scope(f"gwait{t}"):
            for cp in gq[t]:
                cp.wait()
            if pending[ph] is not None:
                for cp in pending[ph]:
                    cp.wait()
        with jax.named_scope(f"combine{t}"):
            combine(rg, ph)
            pending[ph] = write_out(ph, t)
        if t + 3 < NPCH:
            with jax.named_scope(f"idx{t+3}"):
                compute_idx(rg, t + 3)
                gq.append(fire_gathers(rg))
    with jax.named_scope("drain"):
        for ph in range(2):
            if pending[ph] is not None:
                for cp in pending[ph]:
                    cp.wait()


def kernel(img, fix_loc, fixation_size, sampling_grid):
    img_flat = img.reshape(-1)
    gx = sampling_grid[:, 0]
    gy = sampling_grid[:, 1]
    fs_rep = jnp.repeat(fixation_size, L)
    flx_rep = jnp.repeat(fix_loc[:, 0], L)
    fly_rep = jnp.repeat(fix_loc[:, 1], L)
    cx = fix_loc[:, 0] * (W / 2.0) + (W - 1) / 2.0
    cy = fix_loc[:, 1] * (H / 2.0) + (H - 1) / 2.0
    px0 = ((jnp.floor(cx).astype(jnp.int32) - PMARG) // 8) * 8
    py0 = ((jnp.floor(cy).astype(jnp.int32) - PMARG) // 8) * 8
    pxy = jnp.concatenate(
        [jnp.repeat(px0, L)[:, None].reshape(B, L),
         jnp.repeat(py0, L)[:, None].reshape(B, L)], axis=1
    ).reshape(-1).astype(jnp.float32)
    out = _sampler(img_flat, gx, gy, fs_rep, flx_rep, fly_rep, pxy)
    return out.reshape(B, C, N)
